# baseline (speedup 1.0000x reference)
import functools

import jax
import jax.numpy as jnp
from jax import lax
from jax.experimental import pallas as pl
from jax.experimental.pallas import tpu as pltpu
from jax.experimental.pallas import tpu_sc as plsc

_SZ = 4096
_NJB = _SZ // 128
_TEMP = 10.0
_EPS = 1e-20
_LN2 = 0.6931471805599453

_A0 = -1.6853429614838418
_A1 = 2.0280144422257624
_A2 = -0.3426714807419209


def _log2_f32(x):
    b = lax.bitcast_convert_type(x, jnp.int32)
    ef = lax.shift_right_logical(b, 23).astype(jnp.float32)
    m = lax.bitcast_convert_type(
        (b & 0x7FFFFF) | 0x3F800000, jnp.float32)
    p = jnp.float32(_A2)
    p = p * m + jnp.float32(_A1)
    p = p * m + jnp.float32(_A0)
    return (ef - jnp.float32(127.0)) + p


def _glog2(u):
    y = _log2_f32(u + jnp.float32(_EPS))
    return _log2_f32(jnp.float32(0.0) - y)


_C_L = 1.0 / (_TEMP * _LN2)
_C_T = 1.0 / _TEMP


_S_SC = 1024
_NC = 2
_NS = 16
_NW = _NC * _NS
_RB = 8
_JBB = 16
_NBU = (_S_SC // _RB) * (_NJB // _JBB)
_BU_W = _NBU // _NW
_GRP = _RB * _JBB * 8


@functools.partial(
    pl.kernel,
    out_type=jax.ShapeDtypeStruct((_S_SC, _SZ), jnp.float32),
    mesh=plsc.VectorSubcoreMesh(core_axis_name="c", subcore_axis_name="s"),
    scratch_types=[
        pltpu.VMEM((_RB, _JBB, 2, 128), jnp.float32),
        pltpu.VMEM((_RB, _JBB, 2, 128), jnp.float32),
        pltpu.VMEM((_RB, _JBB * 128), jnp.float32),
    ],
    compiler_params=pltpu.CompilerParams(needs_layout_passes=False),
)
def _gumbel_sc(gm_hbm, u_hbm, out_hbm, g_v, u_v, o_v):
    wid = lax.axis_index("s") * _NC + lax.axis_index("c")

    def block(b, carry):
        bu = wid + b * _NW
        r0 = (bu // (_NJB // _JBB)) * _RB
        jb0 = (bu % (_NJB // _JBB)) * _JBB
        pltpu.sync_copy(gm_hbm.at[pl.ds(r0, _RB), pl.ds(jb0, _JBB)], g_v)
        pltpu.sync_copy(u_hbm.at[pl.ds(r0, _RB), pl.ds(jb0, _JBB)], u_v)

        @plsc.parallel_loop(0, _GRP, unroll=4)
        def grp(g):
            r = lax.shift_right_logical(g, 7)
            rem = g & 127
            jb = lax.shift_right_logical(rem, 3)
            jw = (rem & 7) * 16
            l0 = g_v[r, jb, 0, pl.ds(jw, 16)]
            l1 = g_v[r, jb, 1, pl.ds(jw, 16)]
            t0 = _glog2(u_v[r, jb, 0, pl.ds(jw, 16)])
            t1 = _glog2(u_v[r, jb, 1, pl.ds(jw, 16)])
            z2 = (l1 - l0) * jnp.float32(_C_L) + (t0 - t1) * jnp.float32(_C_T)
            s = 1.0 / (1.0 + jnp.exp(z2 * jnp.float32(_LN2)))
            o_v[r, pl.ds(jb * 128 + jw, 16)] = s

        pltpu.sync_copy(o_v, out_hbm.at[pl.ds(r0, _RB), pl.ds(jb0 * 128, _JBB * 128)])
        return carry

    lax.fori_loop(0, _BU_W, block, 0)


_RB_TC = 64
_OFF_TC = _S_SC // _RB_TC


def _tc_body(g_ref, u_ref, o_ref):
    l0 = g_ref[:, 0::2, :]
    l1 = g_ref[:, 1::2, :]
    t0 = _glog2(u_ref[:, 0::2, :])
    t1 = _glog2(u_ref[:, 1::2, :])
    z2 = (l1 - l0) * jnp.float32(_C_L) + (t0 - t1) * jnp.float32(_C_T)
    s = 1.0 / (1.0 + jnp.exp2(z2))
    for jb in range(_NJB):
        o_ref[:, jb * 128:(jb + 1) * 128] = s[:, jb, :]


def kernel(gen_matrix, u):
    gv4 = gen_matrix.reshape(_SZ, _NJB, 128, 2).transpose(0, 1, 3, 2)
    uv4 = u.reshape(_SZ, _NJB, 128, 2).transpose(0, 1, 3, 2)
    gv3 = gv4.reshape(_SZ, 2 * _NJB, 128)
    uv3 = uv4.reshape(_SZ, 2 * _NJB, 128)
    sc_out = _gumbel_sc(gv4, uv4)
    tc_full = pl.pallas_call(
        _tc_body,
        out_shape=jax.ShapeDtypeStruct((_SZ, _SZ), jnp.float32),
        grid=((_SZ - _S_SC) // _RB_TC,),
        in_specs=[
            pl.BlockSpec((_RB_TC, 2 * _NJB, 128),
                         lambda i: (i + _OFF_TC, 0, 0)),
            pl.BlockSpec((_RB_TC, 2 * _NJB, 128),
                         lambda i: (i + _OFF_TC, 0, 0)),
        ],
        out_specs=pl.BlockSpec((_RB_TC, _SZ), lambda i: (i + _OFF_TC, 0)),
    )(gv3, uv3)
    return lax.dynamic_update_slice(tc_full, sc_out, (0, 0))


# speedup vs baseline: 1.0464x; 1.0464x over previous
import functools

import jax
import jax.numpy as jnp
from jax import lax
from jax.experimental import pallas as pl
from jax.experimental.pallas import tpu as pltpu
from jax.experimental.pallas import tpu_sc as plsc

_SZ = 4096
_NJB = _SZ // 128
_TEMP = 10.0
_EPS = 1e-20
_LN2 = 0.6931471805599453

_A0 = -1.6853429614838418
_A1 = 2.0280144422257624
_A2 = -0.3426714807419209


def _log2_f32(x):
    b = lax.bitcast_convert_type(x, jnp.int32)
    ef = lax.shift_right_logical(b, 23).astype(jnp.float32)
    m = lax.bitcast_convert_type(
        (b & 0x7FFFFF) | 0x3F800000, jnp.float32)
    p = jnp.float32(_A2)
    p = p * m + jnp.float32(_A1)
    p = p * m + jnp.float32(_A0)
    return (ef - jnp.float32(127.0)) + p


def _glog2(u):
    y = _log2_f32(u + jnp.float32(_EPS))
    return _log2_f32(jnp.float32(0.0) - y)


_C_L = 1.0 / (_TEMP * _LN2)
_C_T = 1.0 / _TEMP


_S_SC = 896
_NC = 2
_NS = 16
_NW = _NC * _NS
_RB = 8
_JBB = 16
_NBU = (_S_SC // _RB) * (_NJB // _JBB)
_BU_W = _NBU // _NW
_GRP = _RB * _JBB * 8


@functools.partial(
    pl.kernel,
    out_type=jax.ShapeDtypeStruct((_S_SC, _SZ), jnp.float32),
    mesh=plsc.VectorSubcoreMesh(core_axis_name="c", subcore_axis_name="s"),
    scratch_types=[
        pltpu.VMEM((_RB, _JBB, 2, 128), jnp.float32),
        pltpu.VMEM((_RB, _JBB, 2, 128), jnp.float32),
        pltpu.VMEM((_RB, _JBB * 128), jnp.float32),
    ],
    compiler_params=pltpu.CompilerParams(needs_layout_passes=False),
)
def _gumbel_sc(gm_hbm, u_hbm, out_hbm, g_v, u_v, o_v):
    wid = lax.axis_index("s") * _NC + lax.axis_index("c")

    def block(b, carry):
        bu = wid + b * _NW
        r0 = (bu // (_NJB // _JBB)) * _RB
        jb0 = (bu % (_NJB // _JBB)) * _JBB
        pltpu.sync_copy(gm_hbm.at[pl.ds(r0, _RB), pl.ds(jb0, _JBB)], g_v)
        pltpu.sync_copy(u_hbm.at[pl.ds(r0, _RB), pl.ds(jb0, _JBB)], u_v)

        @plsc.parallel_loop(0, _GRP, unroll=4)
        def grp(g):
            r = lax.shift_right_logical(g, 7)
            rem = g & 127
            jb = lax.shift_right_logical(rem, 3)
            jw = (rem & 7) * 16
            l0 = g_v[r, jb, 0, pl.ds(jw, 16)]
            l1 = g_v[r, jb, 1, pl.ds(jw, 16)]
            t0 = _glog2(u_v[r, jb, 0, pl.ds(jw, 16)])
            t1 = _glog2(u_v[r, jb, 1, pl.ds(jw, 16)])
            z2 = (l1 - l0) * jnp.float32(_C_L) + (t0 - t1) * jnp.float32(_C_T)
            s = 1.0 / (1.0 + jnp.exp(z2 * jnp.float32(_LN2)))
            o_v[r, pl.ds(jb * 128 + jw, 16)] = s

        pltpu.sync_copy(o_v, out_hbm.at[pl.ds(r0, _RB), pl.ds(jb0 * 128, _JBB * 128)])
        return carry

    lax.fori_loop(0, _BU_W, block, 0)


_RB_TC = 128
_OFF_TC = _S_SC // _RB_TC


def _tc_body(g_ref, u_ref, o_ref):
    l0 = g_ref[:, 0::2, :]
    l1 = g_ref[:, 1::2, :]
    t0 = _glog2(u_ref[:, 0::2, :])
    t1 = _glog2(u_ref[:, 1::2, :])
    z2 = (l1 - l0) * jnp.float32(_C_L) + (t0 - t1) * jnp.float32(_C_T)
    s = 1.0 / (1.0 + jnp.exp2(z2))
    for jb in range(_NJB):
        o_ref[:, jb * 128:(jb + 1) * 128] = s[:, jb, :]


def kernel(gen_matrix, u):
    gv4 = gen_matrix.reshape(_SZ, _NJB, 128, 2).transpose(0, 1, 3, 2)
    uv4 = u.reshape(_SZ, _NJB, 128, 2).transpose(0, 1, 3, 2)
    gv3 = gv4.reshape(_SZ, 2 * _NJB, 128)
    uv3 = uv4.reshape(_SZ, 2 * _NJB, 128)
    sc_out = _gumbel_sc(gv4, uv4)
    tc_full = pl.pallas_call(
        _tc_body,
        out_shape=jax.ShapeDtypeStruct((_SZ, _SZ), jnp.float32),
        grid=((_SZ - _S_SC) // _RB_TC,),
        in_specs=[
            pl.BlockSpec((_RB_TC, 2 * _NJB, 128),
                         lambda i: (i + _OFF_TC, 0, 0)),
            pl.BlockSpec((_RB_TC, 2 * _NJB, 128),
                         lambda i: (i + _OFF_TC, 0, 0)),
        ],
        out_specs=pl.BlockSpec((_RB_TC, _SZ), lambda i: (i + _OFF_TC, 0)),
    )(gv3, uv3)
    return lax.dynamic_update_slice(tc_full, sc_out, (0, 0))


# speedup vs baseline: 1.0800x; 1.0321x over previous
import functools

import jax
import jax.numpy as jnp
from jax import lax
from jax.experimental import pallas as pl
from jax.experimental.pallas import tpu as pltpu
from jax.experimental.pallas import tpu_sc as plsc

_SZ = 4096
_NJB = _SZ // 128
_TEMP = 10.0
_EPS = 1e-20
_LN2 = 0.6931471805599453

_A0 = -1.6853429614838418
_A1 = 2.0280144422257624
_A2 = -0.3426714807419209


def _log2_f32(x):
    b = lax.bitcast_convert_type(x, jnp.int32)
    ef = lax.shift_right_logical(b, 23).astype(jnp.float32)
    m = lax.bitcast_convert_type(
        (b & 0x7FFFFF) | 0x3F800000, jnp.float32)
    p = jnp.float32(_A2)
    p = p * m + jnp.float32(_A1)
    p = p * m + jnp.float32(_A0)
    return (ef - jnp.float32(127.0)) + p


def _glog2(u):
    y = _log2_f32(u + jnp.float32(_EPS))
    return _log2_f32(jnp.float32(0.0) - y)


_C_L = 1.0 / (_TEMP * _LN2)
_C_T = 1.0 / _TEMP


_S_SC = 768
_NC = 2
_NS = 16
_NW = _NC * _NS
_RB = 8
_JBB = 16
_NBU = (_S_SC // _RB) * (_NJB // _JBB)
_BU_W = _NBU // _NW
_GRP = _RB * _JBB * 8


@functools.partial(
    pl.kernel,
    out_type=jax.ShapeDtypeStruct((_S_SC, _SZ), jnp.float32),
    mesh=plsc.VectorSubcoreMesh(core_axis_name="c", subcore_axis_name="s"),
    scratch_types=[
        pltpu.VMEM((_RB, _JBB, 2, 128), jnp.float32),
        pltpu.VMEM((_RB, _JBB, 2, 128), jnp.float32),
        pltpu.VMEM((_RB, _JBB * 128), jnp.float32),
    ],
    compiler_params=pltpu.CompilerParams(needs_layout_passes=False),
)
def _gumbel_sc(gm_hbm, u_hbm, out_hbm, g_v, u_v, o_v):
    wid = lax.axis_index("s") * _NC + lax.axis_index("c")

    def block(b, carry):
        bu = wid + b * _NW
        r0 = (bu // (_NJB // _JBB)) * _RB
        jb0 = (bu % (_NJB // _JBB)) * _JBB
        pltpu.sync_copy(gm_hbm.at[pl.ds(r0, _RB), pl.ds(jb0, _JBB)], g_v)
        pltpu.sync_copy(u_hbm.at[pl.ds(r0, _RB), pl.ds(jb0, _JBB)], u_v)

        @plsc.parallel_loop(0, _GRP, unroll=4)
        def grp(g):
            r = lax.shift_right_logical(g, 7)
            rem = g & 127
            jb = lax.shift_right_logical(rem, 3)
            jw = (rem & 7) * 16
            l0 = g_v[r, jb, 0, pl.ds(jw, 16)]
            l1 = g_v[r, jb, 1, pl.ds(jw, 16)]
            t0 = _glog2(u_v[r, jb, 0, pl.ds(jw, 16)])
            t1 = _glog2(u_v[r, jb, 1, pl.ds(jw, 16)])
            z2 = (l1 - l0) * jnp.float32(_C_L) + (t0 - t1) * jnp.float32(_C_T)
            s = 1.0 / (1.0 + jnp.exp(z2 * jnp.float32(_LN2)))
            o_v[r, pl.ds(jb * 128 + jw, 16)] = s

        pltpu.sync_copy(o_v, out_hbm.at[pl.ds(r0, _RB), pl.ds(jb0 * 128, _JBB * 128)])
        return carry

    lax.fori_loop(0, _BU_W, block, 0)


_RB_TC = 256
_OFF_TC = _S_SC // _RB_TC


def _tc_body(g_ref, u_ref, o_ref):
    l0 = g_ref[:, 0::2, :]
    l1 = g_ref[:, 1::2, :]
    t0 = _glog2(u_ref[:, 0::2, :])
    t1 = _glog2(u_ref[:, 1::2, :])
    z2 = (l1 - l0) * jnp.float32(_C_L) + (t0 - t1) * jnp.float32(_C_T)
    s = 1.0 / (1.0 + jnp.exp2(z2))
    for jb in range(_NJB):
        o_ref[:, jb * 128:(jb + 1) * 128] = s[:, jb, :]


def kernel(gen_matrix, u):
    gv4 = gen_matrix.reshape(_SZ, _NJB, 128, 2).transpose(0, 1, 3, 2)
    uv4 = u.reshape(_SZ, _NJB, 128, 2).transpose(0, 1, 3, 2)
    gv3 = gv4.reshape(_SZ, 2 * _NJB, 128)
    uv3 = uv4.reshape(_SZ, 2 * _NJB, 128)
    sc_out = _gumbel_sc(gv4, uv4)
    tc_full = pl.pallas_call(
        _tc_body,
        out_shape=jax.ShapeDtypeStruct((_SZ, _SZ), jnp.float32),
        grid=((_SZ - _S_SC) // _RB_TC,),
        in_specs=[
            pl.BlockSpec((_RB_TC, 2 * _NJB, 128),
                         lambda i: (i + _OFF_TC, 0, 0)),
            pl.BlockSpec((_RB_TC, 2 * _NJB, 128),
                         lambda i: (i + _OFF_TC, 0, 0)),
        ],
        out_specs=pl.BlockSpec((_RB_TC, _SZ), lambda i: (i + _OFF_TC, 0)),
    )(gv3, uv3)
    return lax.dynamic_update_slice(tc_full, sc_out, (0, 0))
